# Initial kernel scaffold; baseline (speedup 1.0000x reference)
import jax
import jax.numpy as jnp
from jax.experimental import pallas as pl


def kernel(x):
    raise NotImplementedError("write your pallas kernel here")


# speedup vs baseline: 3.5548x; 3.5548x over previous
import functools

import jax
import jax.numpy as jnp
from jax import lax
from jax.experimental import pallas as pl
from jax.experimental.pallas import tpu as pltpu
from jax.experimental.pallas import tpu_sc as plsc

N = 500000
SIZE = 16
COLS = 2 * SIZE
LANES = 16
NC = 2
NS = 16
NW = NC * NS

CHUNK_ROWS = 800
CHUNK_BLKS = CHUNK_ROWS // LANES
CHUNK_WORDS = CHUNK_ROWS * COLS
TOTAL_CHUNKS = N // CHUNK_ROWS
BASE_CHUNKS = TOTAL_CHUNKS // NW
EXTRA = TOTAL_CHUNKS % NW
MAX_PAIRS = (BASE_CHUNKS + 1 + 1) // 2

X_PAD = N + LANES


def _body(x_hbm, th_hbm, out_hbm, thb, xb0, xb1, ob0, ob1,
          xsem0, xsem1, osem0, osem1):
    cid = lax.axis_index("c")
    sid = lax.axis_index("s")
    wid = sid * NC + cid
    c_lo = wid * BASE_CHUNKS + jnp.minimum(wid, EXTRA)
    n_my = BASE_CHUNKS + (wid < EXTRA).astype(jnp.int32)
    c_hi = c_lo + n_my

    pltpu.sync_copy(th_hbm, thb)
    thv = thb[pl.ds(0, LANES)]
    ths = [jnp.broadcast_to(thv[j], (LANES,)) for j in range(SIZE)]
    ones = jnp.full((LANES,), 1.0, jnp.float32)
    zeros = jnp.full((LANES,), 0.0, jnp.float32)
    iota32 = lax.iota(jnp.int32, LANES) * COLS

    pltpu.async_copy(x_hbm.at[pl.ds(c_lo * CHUNK_ROWS, CHUNK_ROWS + LANES)],
                     xb0, xsem0)
    pltpu.async_copy(x_hbm.at[pl.ds((c_lo + 1) * CHUNK_ROWS, CHUNK_ROWS + LANES)],
                     xb1, xsem1)

    def process(c, xb, ob, xsem, osem):
        pltpu.make_async_copy(
            x_hbm.at[pl.ds(0, CHUNK_ROWS + LANES)], xb, xsem).wait()

        @pl.when(c >= c_lo + 2)
        def _():
            pltpu.make_async_copy(
                ob, out_hbm.at[pl.ds(0, CHUNK_WORDS)], osem).wait()

        def blk(b, carry):
            r0 = b * LANES
            xp = xb[pl.ds(r0, LANES)]
            xn = xb[pl.ds(r0 + 1, LANES)]
            rowbase = iota32 + r0 * COLS
            for j in range(SIZE):
                a = xp <= ths[j]
                bn = xn <= ths[j]
                up = jnp.where(a > bn, ones, zeros)
                dn = jnp.where(bn > a, ones, zeros)
                plsc.store_scatter(ob, [rowbase + (2 * j)], up)
                plsc.store_scatter(ob, [rowbase + (2 * j + 1)], dn)
            return carry

        lax.fori_loop(0, CHUNK_BLKS, blk, 0)

        pltpu.async_copy(ob, out_hbm.at[pl.ds(c * CHUNK_WORDS, CHUNK_WORDS)],
                         osem)

        @pl.when(c + 2 < c_hi)
        def _():
            pltpu.async_copy(
                x_hbm.at[pl.ds((c + 2) * CHUNK_ROWS, CHUNK_ROWS + LANES)],
                xb, xsem)

    def outer(i, carry):
        for s, (xb, ob, xsem, osem) in enumerate(
                ((xb0, ob0, xsem0, osem0), (xb1, ob1, xsem1, osem1))):
            c = c_lo + 2 * i + s

            @pl.when(c < c_hi)
            def _():
                process(c, xb, ob, xsem, osem)
        return carry

    lax.fori_loop(0, MAX_PAIRS, outer, 0)

    pltpu.make_async_copy(ob0, out_hbm.at[pl.ds(0, CHUNK_WORDS)], osem0).wait()
    pltpu.make_async_copy(ob1, out_hbm.at[pl.ds(0, CHUNK_WORDS)], osem1).wait()


@jax.jit
def kernel(x):
    th = jnp.linspace(-1.0, 1.0, SIZE + 2)[1:-1].astype(jnp.float32)
    x_pad = jnp.concatenate([x, jnp.broadcast_to(x[-1], (LANES,))])
    run = functools.partial(
        pl.kernel,
        out_type=jax.ShapeDtypeStruct((N * COLS,), jnp.float32),
        mesh=plsc.VectorSubcoreMesh(core_axis_name="c", subcore_axis_name="s"),
        compiler_params=pltpu.CompilerParams(needs_layout_passes=False),
        scratch_types=[
            pltpu.VMEM((SIZE,), jnp.float32),
            pltpu.VMEM((CHUNK_ROWS + LANES,), jnp.float32),
            pltpu.VMEM((CHUNK_ROWS + LANES,), jnp.float32),
            pltpu.VMEM((CHUNK_WORDS,), jnp.float32),
            pltpu.VMEM((CHUNK_WORDS,), jnp.float32),
            pltpu.SemaphoreType.DMA,
            pltpu.SemaphoreType.DMA,
            pltpu.SemaphoreType.DMA,
            pltpu.SemaphoreType.DMA,
        ],
    )(_body)
    out_flat = run(x_pad, th)
    return out_flat.reshape(N, COLS)


# speedup vs baseline: 5.5916x; 1.5730x over previous
import functools

import jax
import jax.numpy as jnp
from jax import lax
from jax.experimental import pallas as pl
from jax.experimental.pallas import tpu as pltpu
from jax.experimental.pallas import tpu_sc as plsc

N = 500000
SIZE = 16
COLS = 2 * SIZE
LANES = 16
NC = 2
NS = 16
NW = NC * NS

CHUNK_ROWS = 800
CHUNK_BLKS = CHUNK_ROWS // LANES
CHUNK_WORDS = CHUNK_ROWS * COLS
TOTAL_CHUNKS = N // CHUNK_ROWS
BASE_CHUNKS = TOTAL_CHUNKS // NW
EXTRA = TOTAL_CHUNKS % NW
MAX_PAIRS = (BASE_CHUNKS + 1 + 1) // 2

X_PAD = N + LANES


def _body(x_hbm, th_hbm, out_hbm, thb, xb0, xb1, ob0, ob1,
          xsem0, xsem1, osem0, osem1):
    cid = lax.axis_index("c")
    sid = lax.axis_index("s")
    wid = sid * NC + cid
    c_lo = wid * BASE_CHUNKS + jnp.minimum(wid, EXTRA)
    n_my = BASE_CHUNKS + (wid < EXTRA).astype(jnp.int32)
    c_hi = c_lo + n_my

    pltpu.sync_copy(th_hbm, thb)
    th_a = thb[pl.ds(0, LANES)]
    th_b = thb[pl.ds(LANES, LANES)]
    ones = jnp.full((LANES,), 1.0, jnp.float32)
    zeros = jnp.full((LANES,), 0.0, jnp.float32)
    par = (lax.iota(jnp.int32, LANES) & 1) == 1

    pltpu.async_copy(x_hbm.at[pl.ds(c_lo * CHUNK_ROWS, CHUNK_ROWS + LANES)],
                     xb0, xsem0)
    pltpu.async_copy(x_hbm.at[pl.ds((c_lo + 1) * CHUNK_ROWS, CHUNK_ROWS + LANES)],
                     xb1, xsem1)

    def process(c, xb, ob, xsem, osem):
        pltpu.make_async_copy(
            x_hbm.at[pl.ds(0, CHUNK_ROWS + LANES)], xb, xsem).wait()

        @pl.when(c >= c_lo + 2)
        def _():
            pltpu.make_async_copy(
                ob, out_hbm.at[pl.ds(0, CHUNK_WORDS)], osem).wait()

        def blk(b, carry):
            r0 = b * LANES
            base = r0 * COLS
            xp = xb[pl.ds(r0, LANES)]
            xn = xb[pl.ds(r0 + 1, LANES)]
            for k in range(LANES):
                av = jnp.broadcast_to(xp[k], (LANES,))
                bv = jnp.broadcast_to(xn[k], (LANES,))
                for h, thh in ((0, th_a), (1, th_b)):
                    a = (av <= thh) ^ par
                    bn = (bv <= thh) ^ par
                    val = jnp.where(a > bn, ones, zeros)
                    ob[pl.ds(base + k * COLS + h * LANES, LANES)] = val
            return carry

        lax.fori_loop(0, CHUNK_BLKS, blk, 0)

        pltpu.async_copy(ob, out_hbm.at[pl.ds(c * CHUNK_WORDS, CHUNK_WORDS)],
                         osem)

        @pl.when(c + 2 < c_hi)
        def _():
            pltpu.async_copy(
                x_hbm.at[pl.ds((c + 2) * CHUNK_ROWS, CHUNK_ROWS + LANES)],
                xb, xsem)

    def outer(i, carry):
        for s, (xb, ob, xsem, osem) in enumerate(
                ((xb0, ob0, xsem0, osem0), (xb1, ob1, xsem1, osem1))):
            c = c_lo + 2 * i + s

            @pl.when(c < c_hi)
            def _():
                process(c, xb, ob, xsem, osem)
        return carry

    lax.fori_loop(0, MAX_PAIRS, outer, 0)

    pltpu.make_async_copy(ob0, out_hbm.at[pl.ds(0, CHUNK_WORDS)], osem0).wait()
    pltpu.make_async_copy(ob1, out_hbm.at[pl.ds(0, CHUNK_WORDS)], osem1).wait()


@jax.jit
def kernel(x):
    th = jnp.repeat(jnp.linspace(-1.0, 1.0, SIZE + 2)[1:-1].astype(jnp.float32), 2)
    x_pad = jnp.concatenate([x, jnp.broadcast_to(x[-1], (LANES,))])
    run = functools.partial(
        pl.kernel,
        out_type=jax.ShapeDtypeStruct((N * COLS,), jnp.float32),
        mesh=plsc.VectorSubcoreMesh(core_axis_name="c", subcore_axis_name="s"),
        compiler_params=pltpu.CompilerParams(needs_layout_passes=False),
        scratch_types=[
            pltpu.VMEM((COLS,), jnp.float32),
            pltpu.VMEM((CHUNK_ROWS + LANES,), jnp.float32),
            pltpu.VMEM((CHUNK_ROWS + LANES,), jnp.float32),
            pltpu.VMEM((CHUNK_WORDS,), jnp.float32),
            pltpu.VMEM((CHUNK_WORDS,), jnp.float32),
            pltpu.SemaphoreType.DMA,
            pltpu.SemaphoreType.DMA,
            pltpu.SemaphoreType.DMA,
            pltpu.SemaphoreType.DMA,
        ],
    )(_body)
    out_flat = run(x_pad, th)
    return out_flat.reshape(N, COLS)


# speedup vs baseline: 34.6100x; 6.1896x over previous
import functools

import jax
import jax.numpy as jnp
from jax import lax
from jax.experimental import pallas as pl
from jax.experimental.pallas import tpu as pltpu
from jax.experimental.pallas import tpu_sc as plsc

N = 500000
SIZE = 16
COLS = 2 * SIZE
LANES = 16
NC = 2
NS = 16
NW = NC * NS

CHUNK_W = 1024
CHUNK_BLKS = CHUNK_W // LANES
FULL_CHUNKS = N // CHUNK_W
TAIL_LO = FULL_CHUNKS * CHUNK_W
TAIL_W = N - TAIL_LO
TAIL_BLKS = TAIL_W // LANES
BASE_CHUNKS = FULL_CHUNKS // NW
EXTRA = FULL_CHUNKS % NW
MAX_PAIRS = (BASE_CHUNKS + 1 + 1) // 2

X_PAD = N + LANES


def _body(x_hbm, th_hbm, out_hbm, thb, xb0, xb1, ob0, ob1, xbt, obt,
          xsem0, xsem1, osem0, osem1):
    cid = lax.axis_index("c")
    sid = lax.axis_index("s")
    wid = sid * NC + cid
    c_lo = wid * BASE_CHUNKS + jnp.minimum(wid, EXTRA)
    n_my = BASE_CHUNKS + (wid < EXTRA).astype(jnp.int32)
    c_hi = c_lo + n_my

    pltpu.sync_copy(th_hbm, thb)
    thv = thb[pl.ds(0, LANES)]
    ths = [jnp.broadcast_to(thv[j], (LANES,)) for j in range(SIZE)]
    ones = jnp.full((LANES,), 1.0, jnp.float32)
    zeros = jnp.full((LANES,), 0.0, jnp.float32)

    def compute_tile(xb, ob, nblk):
        def blk(b, carry):
            r0 = b * LANES
            xp = xb[pl.ds(r0, LANES)]
            xn = xb[pl.ds(r0 + 1, LANES)]
            for j in range(SIZE):
                a = xp <= ths[j]
                bn = xn <= ths[j]
                ob[2 * j, pl.ds(r0, LANES)] = jnp.where(a > bn, ones, zeros)
                ob[2 * j + 1, pl.ds(r0, LANES)] = jnp.where(bn > a, ones, zeros)
            return carry

        lax.fori_loop(0, nblk, blk, 0)

    pltpu.async_copy(x_hbm.at[pl.ds(c_lo * CHUNK_W, CHUNK_W + LANES)],
                     xb0, xsem0)
    pltpu.async_copy(x_hbm.at[pl.ds((c_lo + 1) * CHUNK_W, CHUNK_W + LANES)],
                     xb1, xsem1)

    def process(c, xb, ob, xsem, osem):
        pltpu.make_async_copy(
            x_hbm.at[pl.ds(0, CHUNK_W + LANES)], xb, xsem).wait()

        @pl.when(c >= c_lo + 2)
        def _():
            pltpu.make_async_copy(
                ob, out_hbm.at[:, pl.ds(0, CHUNK_W)], osem).wait()

        compute_tile(xb, ob, CHUNK_BLKS)

        pltpu.async_copy(ob, out_hbm.at[:, pl.ds(c * CHUNK_W, CHUNK_W)], osem)

        @pl.when(c + 2 < c_hi)
        def _():
            pltpu.async_copy(
                x_hbm.at[pl.ds((c + 2) * CHUNK_W, CHUNK_W + LANES)],
                xb, xsem)

    def outer(i, carry):
        for s, (xb, ob, xsem, osem) in enumerate(
                ((xb0, ob0, xsem0, osem0), (xb1, ob1, xsem1, osem1))):
            c = c_lo + 2 * i + s

            @pl.when(c < c_hi)
            def _():
                process(c, xb, ob, xsem, osem)
        return carry

    lax.fori_loop(0, MAX_PAIRS, outer, 0)

    @pl.when(wid == NW - 1)
    def _():
        pltpu.sync_copy(x_hbm.at[pl.ds(TAIL_LO, TAIL_W + LANES)], xbt)
        compute_tile(xbt, obt, TAIL_BLKS)
        pltpu.sync_copy(obt, out_hbm.at[:, pl.ds(TAIL_LO, TAIL_W)])

    pltpu.make_async_copy(ob0, out_hbm.at[:, pl.ds(0, CHUNK_W)], osem0).wait()
    pltpu.make_async_copy(ob1, out_hbm.at[:, pl.ds(0, CHUNK_W)], osem1).wait()


@jax.jit
def kernel(x):
    th = jnp.linspace(-1.0, 1.0, SIZE + 2)[1:-1].astype(jnp.float32)
    x_pad = jnp.concatenate([x, jnp.broadcast_to(x[-1], (LANES,))])
    run = functools.partial(
        pl.kernel,
        out_type=jax.ShapeDtypeStruct((COLS, N), jnp.float32),
        mesh=plsc.VectorSubcoreMesh(core_axis_name="c", subcore_axis_name="s"),
        compiler_params=pltpu.CompilerParams(needs_layout_passes=False),
        scratch_types=[
            pltpu.VMEM((LANES,), jnp.float32),
            pltpu.VMEM((CHUNK_W + LANES,), jnp.float32),
            pltpu.VMEM((CHUNK_W + LANES,), jnp.float32),
            pltpu.VMEM((COLS, CHUNK_W), jnp.float32),
            pltpu.VMEM((COLS, CHUNK_W), jnp.float32),
            pltpu.VMEM((TAIL_W + LANES,), jnp.float32),
            pltpu.VMEM((COLS, TAIL_W), jnp.float32),
            pltpu.SemaphoreType.DMA,
            pltpu.SemaphoreType.DMA,
            pltpu.SemaphoreType.DMA,
            pltpu.SemaphoreType.DMA,
        ],
    )(_body)
    return run(x_pad, th).T


# speedup vs baseline: 36.1942x; 1.0458x over previous
import functools

import jax
import jax.numpy as jnp
import numpy as np
from jax import lax
from jax.experimental import pallas as pl
from jax.experimental.pallas import tpu as pltpu
from jax.experimental.pallas import tpu_sc as plsc

N = 500000
SIZE = 16
COLS = 2 * SIZE
LANES = 16
NC = 2
NS = 16
NW = NC * NS

CHUNK_W = 1024
CHUNK_BLKS = CHUNK_W // LANES
UNROLL = 4
FULL_CHUNKS = N // CHUNK_W
TAIL_LO = FULL_CHUNKS * CHUNK_W
TAIL_W = N - TAIL_LO
TAIL_BLKS = TAIL_W // LANES
BASE_CHUNKS = FULL_CHUNKS // NW
EXTRA = FULL_CHUNKS % NW
MAX_PAIRS = (BASE_CHUNKS + 1 + 1) // 2


def _body(x_hbm, th_hbm, out_hbm, thb, xb0, xb1, ob0, ob1, xbt, obt,
          xsem0, xsem1, osem0, osem1):
    cid = lax.axis_index("c")
    sid = lax.axis_index("s")
    wid = sid * NC + cid
    c_lo = wid * BASE_CHUNKS + jnp.minimum(wid, EXTRA)
    n_my = BASE_CHUNKS + (wid < EXTRA).astype(jnp.int32)
    c_hi = c_lo + n_my

    pltpu.sync_copy(th_hbm, thb)
    thv = thb[pl.ds(0, LANES)]
    ths = [jnp.broadcast_to(thv[j], (LANES,)) for j in range(SIZE)]
    ones = jnp.full((LANES,), 1.0, jnp.float32)
    zeros = jnp.full((LANES,), 0.0, jnp.float32)

    def one_block(xb, ob, r0):
        xp = xb[pl.ds(r0, LANES)]
        xn = xb[pl.ds(r0 + 1, LANES)]
        for j in range(SIZE):
            a = xp <= ths[j]
            bn = xn <= ths[j]
            ob[2 * j, pl.ds(r0, LANES)] = jnp.where(a > bn, ones, zeros)
            ob[2 * j + 1, pl.ds(r0, LANES)] = jnp.where(bn > a, ones, zeros)

    def compute_tile(xb, ob, niter, unroll):
        def blk(b, carry):
            r0 = b * (LANES * unroll)
            for u in range(unroll):
                one_block(xb, ob, r0 + u * LANES)
            return carry

        lax.fori_loop(0, niter, blk, 0)

    pltpu.async_copy(x_hbm.at[pl.ds(c_lo * CHUNK_W, CHUNK_W + LANES)],
                     xb0, xsem0)
    pltpu.async_copy(x_hbm.at[pl.ds((c_lo + 1) * CHUNK_W, CHUNK_W + LANES)],
                     xb1, xsem1)

    def process(c, xb, ob, xsem, osem):
        pltpu.make_async_copy(
            x_hbm.at[pl.ds(0, CHUNK_W + LANES)], xb, xsem).wait()

        @pl.when(c >= c_lo + 2)
        def _():
            pltpu.make_async_copy(
                ob, out_hbm.at[:, pl.ds(0, CHUNK_W)], osem).wait()

        compute_tile(xb, ob, CHUNK_BLKS // UNROLL, UNROLL)

        pltpu.async_copy(ob, out_hbm.at[:, pl.ds(c * CHUNK_W, CHUNK_W)], osem)

        @pl.when(c + 2 < c_hi)
        def _():
            pltpu.async_copy(
                x_hbm.at[pl.ds((c + 2) * CHUNK_W, CHUNK_W + LANES)],
                xb, xsem)

    def outer(i, carry):
        for s, (xb, ob, xsem, osem) in enumerate(
                ((xb0, ob0, xsem0, osem0), (xb1, ob1, xsem1, osem1))):
            c = c_lo + 2 * i + s

            @pl.when(c < c_hi)
            def _():
                process(c, xb, ob, xsem, osem)
        return carry

    lax.fori_loop(0, MAX_PAIRS, outer, 0)

    @pl.when(wid == NW - 1)
    def _():
        pltpu.sync_copy(x_hbm.at[pl.ds(TAIL_LO, TAIL_W)],
                        xbt.at[pl.ds(0, TAIL_W)])
        last = xbt[pl.ds(TAIL_W - LANES, LANES)]
        xbt[pl.ds(TAIL_W, LANES)] = jnp.broadcast_to(last[LANES - 1], (LANES,))
        compute_tile(xbt, obt, TAIL_BLKS // 2, 2)
        pltpu.sync_copy(obt, out_hbm.at[:, pl.ds(TAIL_LO, TAIL_W)])

    pltpu.make_async_copy(ob0, out_hbm.at[:, pl.ds(0, CHUNK_W)], osem0).wait()
    pltpu.make_async_copy(ob1, out_hbm.at[:, pl.ds(0, CHUNK_W)], osem1).wait()


@jax.jit
def kernel(x):
    th = jnp.asarray(np.linspace(-1.0, 1.0, SIZE + 2)[1:-1], jnp.float32)
    run = functools.partial(
        pl.kernel,
        out_type=jax.ShapeDtypeStruct((COLS, N), jnp.float32),
        mesh=plsc.VectorSubcoreMesh(core_axis_name="c", subcore_axis_name="s"),
        compiler_params=pltpu.CompilerParams(needs_layout_passes=False),
        scratch_types=[
            pltpu.VMEM((LANES,), jnp.float32),
            pltpu.VMEM((CHUNK_W + LANES,), jnp.float32),
            pltpu.VMEM((CHUNK_W + LANES,), jnp.float32),
            pltpu.VMEM((COLS, CHUNK_W), jnp.float32),
            pltpu.VMEM((COLS, CHUNK_W), jnp.float32),
            pltpu.VMEM((TAIL_W + LANES,), jnp.float32),
            pltpu.VMEM((COLS, TAIL_W), jnp.float32),
            pltpu.SemaphoreType.DMA,
            pltpu.SemaphoreType.DMA,
            pltpu.SemaphoreType.DMA,
            pltpu.SemaphoreType.DMA,
        ],
    )(_body)
    return run(x, th).T


# speedup vs baseline: 36.4456x; 1.0069x over previous
import functools

import jax
import jax.numpy as jnp
import numpy as np
from jax import lax
from jax.experimental import pallas as pl
from jax.experimental.pallas import tpu as pltpu
from jax.experimental.pallas import tpu_sc as plsc

N = 500000
SIZE = 16
COLS = 2 * SIZE
LANES = 16
NC = 2
NS = 16
NW = NC * NS

CHUNK_W = 1024
CHUNK_BLKS = CHUNK_W // LANES
UNROLL = 2
FULL_CHUNKS = N // CHUNK_W
TAIL_LO = FULL_CHUNKS * CHUNK_W
TAIL_W = N - TAIL_LO
TAIL_BLKS = TAIL_W // LANES
BASE_CHUNKS = FULL_CHUNKS // NW
EXTRA = FULL_CHUNKS % NW
MAX_PAIRS = (BASE_CHUNKS + 1 + 1) // 2


def _body(x_hbm, th_hbm, out_hbm, thb, xb0, xb1, ob0, ob1, xbt, obt,
          xsem0, xsem1, osem0, osem1):
    cid = lax.axis_index("c")
    sid = lax.axis_index("s")
    wid = sid * NC + cid
    c_lo = wid * BASE_CHUNKS + jnp.minimum(wid, EXTRA)
    n_my = BASE_CHUNKS + (wid < EXTRA).astype(jnp.int32)
    c_hi = c_lo + n_my

    pltpu.sync_copy(th_hbm, thb)
    thv = thb[pl.ds(0, LANES)]
    ths = [jnp.broadcast_to(thv[j], (LANES,)) for j in range(SIZE)]
    ones = jnp.full((LANES,), 1.0, jnp.float32)
    zeros = jnp.full((LANES,), 0.0, jnp.float32)

    def one_block(xb, ob, r0):
        xp = xb[pl.ds(r0, LANES)]
        xn = xb[pl.ds(r0 + 1, LANES)]
        for j in range(SIZE):
            a = xp <= ths[j]
            bn = xn <= ths[j]
            ob[2 * j, pl.ds(r0, LANES)] = jnp.where(a > bn, ones, zeros)
            ob[2 * j + 1, pl.ds(r0, LANES)] = jnp.where(bn > a, ones, zeros)

    def compute_tile(xb, ob, niter, unroll):
        def blk(b, carry):
            r0 = b * (LANES * unroll)
            for u in range(unroll):
                one_block(xb, ob, r0 + u * LANES)
            return carry

        lax.fori_loop(0, niter, blk, 0)

    pltpu.async_copy(x_hbm.at[pl.ds(c_lo * CHUNK_W, CHUNK_W + LANES)],
                     xb0, xsem0)
    pltpu.async_copy(x_hbm.at[pl.ds((c_lo + 1) * CHUNK_W, CHUNK_W + LANES)],
                     xb1, xsem1)

    def process(c, xb, ob, xsem, osem):
        pltpu.make_async_copy(
            x_hbm.at[pl.ds(0, CHUNK_W + LANES)], xb, xsem).wait()

        @pl.when(c >= c_lo + 2)
        def _():
            pltpu.make_async_copy(
                ob, out_hbm.at[:, pl.ds(0, CHUNK_W)], osem).wait()

        compute_tile(xb, ob, CHUNK_BLKS // UNROLL, UNROLL)

        pltpu.async_copy(ob, out_hbm.at[:, pl.ds(c * CHUNK_W, CHUNK_W)], osem)

        @pl.when(c + 2 < c_hi)
        def _():
            pltpu.async_copy(
                x_hbm.at[pl.ds((c + 2) * CHUNK_W, CHUNK_W + LANES)],
                xb, xsem)

    def outer(i, carry):
        for s, (xb, ob, xsem, osem) in enumerate(
                ((xb0, ob0, xsem0, osem0), (xb1, ob1, xsem1, osem1))):
            c = c_lo + 2 * i + s

            @pl.when(c < c_hi)
            def _():
                process(c, xb, ob, xsem, osem)
        return carry

    lax.fori_loop(0, MAX_PAIRS, outer, 0)

    @pl.when(wid == NW - 1)
    def _():
        pltpu.sync_copy(x_hbm.at[pl.ds(TAIL_LO, TAIL_W)],
                        xbt.at[pl.ds(0, TAIL_W)])
        last = xbt[pl.ds(TAIL_W - LANES, LANES)]
        xbt[pl.ds(TAIL_W, LANES)] = jnp.broadcast_to(last[LANES - 1], (LANES,))
        compute_tile(xbt, obt, TAIL_BLKS // 2, 2)
        pltpu.sync_copy(obt, out_hbm.at[:, pl.ds(TAIL_LO, TAIL_W)])

    pltpu.make_async_copy(ob0, out_hbm.at[:, pl.ds(0, CHUNK_W)], osem0).wait()
    pltpu.make_async_copy(ob1, out_hbm.at[:, pl.ds(0, CHUNK_W)], osem1).wait()


@jax.jit
def kernel(x):
    th = jnp.asarray(np.linspace(-1.0, 1.0, SIZE + 2)[1:-1], jnp.float32)
    run = functools.partial(
        pl.kernel,
        out_type=jax.ShapeDtypeStruct((COLS, N), jnp.float32),
        mesh=plsc.VectorSubcoreMesh(core_axis_name="c", subcore_axis_name="s"),
        compiler_params=pltpu.CompilerParams(needs_layout_passes=False),
        scratch_types=[
            pltpu.VMEM((LANES,), jnp.float32),
            pltpu.VMEM((CHUNK_W + LANES,), jnp.float32),
            pltpu.VMEM((CHUNK_W + LANES,), jnp.float32),
            pltpu.VMEM((COLS, CHUNK_W), jnp.float32),
            pltpu.VMEM((COLS, CHUNK_W), jnp.float32),
            pltpu.VMEM((TAIL_W + LANES,), jnp.float32),
            pltpu.VMEM((COLS, TAIL_W), jnp.float32),
            pltpu.SemaphoreType.DMA,
            pltpu.SemaphoreType.DMA,
            pltpu.SemaphoreType.DMA,
            pltpu.SemaphoreType.DMA,
        ],
    )(_body)
    return run(x, th).T


# speedup vs baseline: 36.4924x; 1.0013x over previous
import functools

import jax
import jax.numpy as jnp
import numpy as np
from jax import lax
from jax.experimental import pallas as pl
from jax.experimental.pallas import tpu as pltpu
from jax.experimental.pallas import tpu_sc as plsc

N = 500000
SIZE = 16
COLS = 2 * SIZE
LANES = 16
NC = 2
NS = 16
NW = NC * NS

CHUNK_W = 1024
CHUNK_BLKS = CHUNK_W // LANES
UNROLL = 1
FULL_CHUNKS = N // CHUNK_W
TAIL_LO = FULL_CHUNKS * CHUNK_W
TAIL_W = N - TAIL_LO
TAIL_BLKS = TAIL_W // LANES
BASE_CHUNKS = FULL_CHUNKS // NW
EXTRA = FULL_CHUNKS % NW
MAX_PAIRS = (BASE_CHUNKS + 1 + 1) // 2


def _body(x_hbm, th_hbm, out_hbm, thb, xb0, xb1, ob0, ob1, xbt, obt,
          xsem0, xsem1, osem0, osem1):
    cid = lax.axis_index("c")
    sid = lax.axis_index("s")
    wid = sid * NC + cid
    c_lo = wid * BASE_CHUNKS + jnp.minimum(wid, EXTRA)
    n_my = BASE_CHUNKS + (wid < EXTRA).astype(jnp.int32)
    c_hi = c_lo + n_my

    pltpu.sync_copy(th_hbm, thb)
    thv = thb[pl.ds(0, LANES)]
    ths = [jnp.broadcast_to(thv[j], (LANES,)) for j in range(SIZE)]
    ones = jnp.full((LANES,), 1.0, jnp.float32)
    zeros = jnp.full((LANES,), 0.0, jnp.float32)

    def one_block(xb, ob, r0):
        xp = xb[pl.ds(r0, LANES)]
        xn = xb[pl.ds(r0 + 1, LANES)]
        for j in range(SIZE):
            a = xp <= ths[j]
            bn = xn <= ths[j]
            ob[2 * j, pl.ds(r0, LANES)] = jnp.where(a > bn, ones, zeros)
            ob[2 * j + 1, pl.ds(r0, LANES)] = jnp.where(bn > a, ones, zeros)

    def compute_tile(xb, ob, niter, unroll):
        def blk(b, carry):
            r0 = b * (LANES * unroll)
            for u in range(unroll):
                one_block(xb, ob, r0 + u * LANES)
            return carry

        lax.fori_loop(0, niter, blk, 0)

    pltpu.async_copy(x_hbm.at[pl.ds(c_lo * CHUNK_W, CHUNK_W + LANES)],
                     xb0, xsem0)
    pltpu.async_copy(x_hbm.at[pl.ds((c_lo + 1) * CHUNK_W, CHUNK_W + LANES)],
                     xb1, xsem1)

    def process(c, xb, ob, xsem, osem):
        pltpu.make_async_copy(
            x_hbm.at[pl.ds(0, CHUNK_W + LANES)], xb, xsem).wait()

        @pl.when(c >= c_lo + 2)
        def _():
            pltpu.make_async_copy(
                ob, out_hbm.at[:, pl.ds(0, CHUNK_W)], osem).wait()

        compute_tile(xb, ob, CHUNK_BLKS // UNROLL, UNROLL)

        pltpu.async_copy(ob, out_hbm.at[:, pl.ds(c * CHUNK_W, CHUNK_W)], osem)

        @pl.when(c + 2 < c_hi)
        def _():
            pltpu.async_copy(
                x_hbm.at[pl.ds((c + 2) * CHUNK_W, CHUNK_W + LANES)],
                xb, xsem)

    def outer(i, carry):
        for s, (xb, ob, xsem, osem) in enumerate(
                ((xb0, ob0, xsem0, osem0), (xb1, ob1, xsem1, osem1))):
            c = c_lo + 2 * i + s

            @pl.when(c < c_hi)
            def _():
                process(c, xb, ob, xsem, osem)
        return carry

    lax.fori_loop(0, MAX_PAIRS, outer, 0)

    @pl.when(wid == NW - 1)
    def _():
        pltpu.sync_copy(x_hbm.at[pl.ds(TAIL_LO, TAIL_W)],
                        xbt.at[pl.ds(0, TAIL_W)])
        last = xbt[pl.ds(TAIL_W - LANES, LANES)]
        xbt[pl.ds(TAIL_W, LANES)] = jnp.broadcast_to(last[LANES - 1], (LANES,))
        compute_tile(xbt, obt, TAIL_BLKS // 2, 2)
        pltpu.sync_copy(obt, out_hbm.at[:, pl.ds(TAIL_LO, TAIL_W)])

    pltpu.make_async_copy(ob0, out_hbm.at[:, pl.ds(0, CHUNK_W)], osem0).wait()
    pltpu.make_async_copy(ob1, out_hbm.at[:, pl.ds(0, CHUNK_W)], osem1).wait()


@jax.jit
def kernel(x):
    th = jnp.asarray(np.linspace(-1.0, 1.0, SIZE + 2)[1:-1], jnp.float32)
    run = functools.partial(
        pl.kernel,
        out_type=jax.ShapeDtypeStruct((COLS, N), jnp.float32),
        mesh=plsc.VectorSubcoreMesh(core_axis_name="c", subcore_axis_name="s"),
        compiler_params=pltpu.CompilerParams(needs_layout_passes=False),
        scratch_types=[
            pltpu.VMEM((LANES,), jnp.float32),
            pltpu.VMEM((CHUNK_W + LANES,), jnp.float32),
            pltpu.VMEM((CHUNK_W + LANES,), jnp.float32),
            pltpu.VMEM((COLS, CHUNK_W), jnp.float32),
            pltpu.VMEM((COLS, CHUNK_W), jnp.float32),
            pltpu.VMEM((TAIL_W + LANES,), jnp.float32),
            pltpu.VMEM((COLS, TAIL_W), jnp.float32),
            pltpu.SemaphoreType.DMA,
            pltpu.SemaphoreType.DMA,
            pltpu.SemaphoreType.DMA,
            pltpu.SemaphoreType.DMA,
        ],
    )(_body)
    return run(x, th).T


# speedup vs baseline: 36.5590x; 1.0018x over previous
import functools

import jax
import jax.numpy as jnp
from jax import lax
from jax.experimental import pallas as pl
from jax.experimental.pallas import tpu as pltpu
from jax.experimental.pallas import tpu_sc as plsc

N = 500000
SIZE = 16
COLS = 2 * SIZE
LANES = 16
NC = 2
NS = 16
NW = NC * NS

CHUNK_W = 1024
CHUNK_BLKS = CHUNK_W // LANES
UNROLL = 1
FULL_CHUNKS = N // CHUNK_W
TAIL_LO = FULL_CHUNKS * CHUNK_W
TAIL_W = N - TAIL_LO
TAIL_BLKS = TAIL_W // LANES
BASE_CHUNKS = FULL_CHUNKS // NW
EXTRA = FULL_CHUNKS % NW
MAX_PAIRS = (BASE_CHUNKS + 1 + 1) // 2


def _body(x_hbm, th_hbm, out_hbm, thb, xb0, xb1, ob0, ob1, xbt, obt,
          xsem0, xsem1, osem0, osem1):
    cid = lax.axis_index("c")
    sid = lax.axis_index("s")
    wid = sid * NC + cid
    c_lo = wid * BASE_CHUNKS + jnp.minimum(wid, EXTRA)
    n_my = BASE_CHUNKS + (wid < EXTRA).astype(jnp.int32)
    c_hi = c_lo + n_my

    pltpu.sync_copy(th_hbm, thb)
    thv = thb[pl.ds(0, LANES)]
    ths = [jnp.broadcast_to(thv[j], (LANES,)) for j in range(SIZE)]
    ones = jnp.full((LANES,), 1.0, jnp.float32)
    zeros = jnp.full((LANES,), 0.0, jnp.float32)

    def one_block(xb, ob, r0):
        xp = xb[pl.ds(r0, LANES)]
        xn = xb[pl.ds(r0 + 1, LANES)]
        for j in range(SIZE):
            a = xp <= ths[j]
            bn = xn <= ths[j]
            ob[2 * j, pl.ds(r0, LANES)] = jnp.where(a > bn, ones, zeros)
            ob[2 * j + 1, pl.ds(r0, LANES)] = jnp.where(bn > a, ones, zeros)

    def compute_tile(xb, ob, niter, unroll):
        def blk(b, carry):
            r0 = b * (LANES * unroll)
            for u in range(unroll):
                one_block(xb, ob, r0 + u * LANES)
            return carry

        lax.fori_loop(0, niter, blk, 0)

    pltpu.async_copy(x_hbm.at[pl.ds(c_lo * CHUNK_W, CHUNK_W + LANES)],
                     xb0, xsem0)
    pltpu.async_copy(x_hbm.at[pl.ds((c_lo + 1) * CHUNK_W, CHUNK_W + LANES)],
                     xb1, xsem1)

    def process(c, xb, ob, xsem, osem):
        pltpu.make_async_copy(
            x_hbm.at[pl.ds(0, CHUNK_W + LANES)], xb, xsem).wait()

        @pl.when(c >= c_lo + 2)
        def _():
            pltpu.make_async_copy(
                ob, out_hbm.at[:, pl.ds(0, CHUNK_W)], osem).wait()

        compute_tile(xb, ob, CHUNK_BLKS // UNROLL, UNROLL)

        pltpu.async_copy(ob, out_hbm.at[:, pl.ds(c * CHUNK_W, CHUNK_W)], osem)

        @pl.when(c + 2 < c_hi)
        def _():
            pltpu.async_copy(
                x_hbm.at[pl.ds((c + 2) * CHUNK_W, CHUNK_W + LANES)],
                xb, xsem)

    def outer(i, carry):
        for s, (xb, ob, xsem, osem) in enumerate(
                ((xb0, ob0, xsem0, osem0), (xb1, ob1, xsem1, osem1))):
            c = c_lo + 2 * i + s

            @pl.when(c < c_hi)
            def _():
                process(c, xb, ob, xsem, osem)
        return carry

    lax.fori_loop(0, MAX_PAIRS, outer, 0)

    @pl.when(wid == NW - 1)
    def _():
        pltpu.sync_copy(x_hbm.at[pl.ds(TAIL_LO, TAIL_W)],
                        xbt.at[pl.ds(0, TAIL_W)])
        last = xbt[pl.ds(TAIL_W - LANES, LANES)]
        xbt[pl.ds(TAIL_W, LANES)] = jnp.broadcast_to(last[LANES - 1], (LANES,))
        compute_tile(xbt, obt, TAIL_BLKS // 2, 2)
        pltpu.sync_copy(obt, out_hbm.at[:, pl.ds(TAIL_LO, TAIL_W)])

    pltpu.make_async_copy(ob0, out_hbm.at[:, pl.ds(0, CHUNK_W)], osem0).wait()
    pltpu.make_async_copy(ob1, out_hbm.at[:, pl.ds(0, CHUNK_W)], osem1).wait()


@jax.jit
def kernel(x):
    th = jnp.linspace(-1.0, 1.0, SIZE + 2)[1:-1].astype(jnp.float32)
    run = functools.partial(
        pl.kernel,
        out_type=jax.ShapeDtypeStruct((COLS, N), jnp.float32),
        mesh=plsc.VectorSubcoreMesh(core_axis_name="c", subcore_axis_name="s"),
        compiler_params=pltpu.CompilerParams(needs_layout_passes=False),
        scratch_types=[
            pltpu.VMEM((LANES,), jnp.float32),
            pltpu.VMEM((CHUNK_W + LANES,), jnp.float32),
            pltpu.VMEM((CHUNK_W + LANES,), jnp.float32),
            pltpu.VMEM((COLS, CHUNK_W), jnp.float32),
            pltpu.VMEM((COLS, CHUNK_W), jnp.float32),
            pltpu.VMEM((TAIL_W + LANES,), jnp.float32),
            pltpu.VMEM((COLS, TAIL_W), jnp.float32),
            pltpu.SemaphoreType.DMA,
            pltpu.SemaphoreType.DMA,
            pltpu.SemaphoreType.DMA,
            pltpu.SemaphoreType.DMA,
        ],
    )(_body)
    return run(x, th).T
